# baseline (speedup 1.0000x reference)
import dataclasses
import functools

import jax
import jax.numpy as jnp
from jax import lax
from jax.experimental import pallas as pl
from jax.experimental.pallas import tpu as pltpu
from jax.experimental.pallas import tpu_sc as plsc

_NC = 2
_NS = 16
_NW = _NC * _NS
_L = 16
_UNROLL = 4


def _sc_table_gather(idx, stab, ctab, *, chunk):
    n = idx.shape[0]
    mesh = plsc.VectorSubcoreMesh(core_axis_name="c", subcore_axis_name="s")
    cp = pltpu.CompilerParams()
    if "needs_layout_passes" in pltpu.CompilerParams.__dataclass_fields__:
        cp = dataclasses.replace(cp, needs_layout_passes=False)

    @functools.partial(
        pl.kernel,
        out_type=(jax.ShapeDtypeStruct((n,), jnp.float32),
                  jax.ShapeDtypeStruct((n,), jnp.float32)),
        mesh=mesh,
        compiler_params=cp,
        scratch_types=[
            pltpu.VMEM((chunk,), jnp.int32),
            pltpu.VMEM((chunk,), jnp.float32),
            pltpu.VMEM((chunk,), jnp.float32),
            pltpu.VMEM(stab.shape, jnp.float32),
            pltpu.VMEM(ctab.shape, jnp.float32),
            pltpu.SemaphoreType.DMA,
            pltpu.SemaphoreType.DMA,
            pltpu.SemaphoreType.DMA,
            pltpu.SemaphoreType.DMA,
        ],
    )
    def body(idx_hbm, stab_hbm, ctab_hbm, s_hbm, sc_hbm,
             idx_v, s_v, sc_v, stab_v, ctab_v,
             sem0, sem1, sem2, sem3):
        wid = lax.axis_index("s") * _NC + lax.axis_index("c")
        base = jnp.where(wid == _NW - 1, n - chunk, wid * chunk)

        c0 = pltpu.async_copy(idx_hbm.at[pl.ds(base, chunk)], idx_v, sem0)
        c2 = pltpu.async_copy(stab_hbm, stab_v, sem2)
        c3 = pltpu.async_copy(ctab_hbm, ctab_v, sem3)
        c0.wait()
        c2.wait()
        c3.wait()

        def vec(c):
            sl = pl.ds(c, _L)
            iv = idx_v[sl]
            s_v[sl] = plsc.load_gather(stab_v, [iv])
            sc_v[sl] = plsc.load_gather(ctab_v, [iv])

        main = chunk - chunk % (_L * _UNROLL)

        @pl.loop(0, main, step=_L * _UNROLL)
        def _(c):
            for u in range(_UNROLL):
                vec(c + u * _L)

        @pl.loop(main, chunk, step=_L)
        def _(c):
            vec(c)

        o0 = pltpu.async_copy(s_v, s_hbm.at[pl.ds(base, chunk)], sem0)
        o1 = pltpu.async_copy(sc_v, sc_hbm.at[pl.ds(base, chunk)], sem1)
        o0.wait()
        o1.wait()

    return body(idx, stab, ctab)


def kernel(in_field, species_idx, shifts, scales):
    n = in_field.shape[0]
    idx = species_idx.astype(jnp.int32)
    stab = shifts.astype(jnp.float32)
    ctab = scales.astype(jnp.float32)

    chunk = ((n + _NW - 1) // _NW + _L - 1) // _L * _L
    assert chunk % _L == 0 and n % 8 == 0 and n >= chunk, (n, chunk)

    s_arr, sc_arr = _sc_table_gather(idx, stab, ctab, chunk=chunk)
    return s_arr.reshape(n, 1) + sc_arr.reshape(n, 1) * in_field


# speedup vs baseline: 1.0019x; 1.0019x over previous
import dataclasses
import functools

import jax
import jax.numpy as jnp
from jax import lax
from jax.experimental import pallas as pl
from jax.experimental.pallas import tpu as pltpu
from jax.experimental.pallas import tpu_sc as plsc

_NC = 2
_NS = 16
_NW = _NC * _NS
_L = 16
_UNROLL = 8


def _sc_affine_gather(x, idx, stab, ctab, *, chunk):
    n = x.shape[0]
    mesh = plsc.VectorSubcoreMesh(core_axis_name="c", subcore_axis_name="s")
    cp = pltpu.CompilerParams()
    if "needs_layout_passes" in pltpu.CompilerParams.__dataclass_fields__:
        cp = dataclasses.replace(cp, needs_layout_passes=False)

    @functools.partial(
        pl.kernel,
        out_type=jax.ShapeDtypeStruct((n,), jnp.float32),
        mesh=mesh,
        compiler_params=cp,
        scratch_types=[
            pltpu.VMEM((chunk,), jnp.int32),
            pltpu.VMEM((chunk,), jnp.float32),
            pltpu.VMEM((chunk,), jnp.float32),
            pltpu.VMEM(stab.shape, jnp.float32),
            pltpu.VMEM(ctab.shape, jnp.float32),
            pltpu.SemaphoreType.DMA,
            pltpu.SemaphoreType.DMA,
            pltpu.SemaphoreType.DMA,
            pltpu.SemaphoreType.DMA,
        ],
    )
    def body(x_hbm, idx_hbm, stab_hbm, ctab_hbm, out_hbm,
             idx_v, x_v, out_v, stab_v, ctab_v,
             sem0, sem1, sem2, sem3):
        wid = lax.axis_index("s") * _NC + lax.axis_index("c")
        base = jnp.where(wid == _NW - 1, n - chunk, wid * chunk)

        c0 = pltpu.async_copy(idx_hbm.at[pl.ds(base, chunk)], idx_v, sem0)
        c1 = pltpu.async_copy(x_hbm.at[pl.ds(base, chunk)], x_v, sem1)
        c2 = pltpu.async_copy(stab_hbm, stab_v, sem2)
        c3 = pltpu.async_copy(ctab_hbm, ctab_v, sem3)
        c0.wait()
        c1.wait()
        c2.wait()
        c3.wait()

        def vec(c):
            sl = pl.ds(c, _L)
            iv = idx_v[sl]
            xv = x_v[sl]
            s = plsc.load_gather(stab_v, [iv])
            sc = plsc.load_gather(ctab_v, [iv])
            out_v[sl] = s + sc * xv

        main = chunk - chunk % (_L * _UNROLL)

        @pl.loop(0, main, step=_L * _UNROLL)
        def _(c):
            for u in range(_UNROLL):
                vec(c + u * _L)

        @pl.loop(main, chunk, step=_L)
        def _(c):
            vec(c)

        pltpu.sync_copy(out_v, out_hbm.at[pl.ds(base, chunk)])

    return body(x, idx, stab, ctab)


def kernel(in_field, species_idx, shifts, scales):
    n = in_field.shape[0]
    x = in_field.reshape(n).astype(jnp.float32)
    idx = species_idx.astype(jnp.int32)
    stab = shifts.astype(jnp.float32)
    ctab = scales.astype(jnp.float32)

    chunk = ((n + _NW - 1) // _NW + _L - 1) // _L * _L
    assert chunk % _L == 0 and n % 8 == 0 and n >= chunk, (n, chunk)

    out = _sc_affine_gather(x, idx, stab, ctab, chunk=chunk)
    return out.reshape(n, 1)


# speedup vs baseline: 1.0179x; 1.0159x over previous
import dataclasses
import functools

import jax
import jax.numpy as jnp
from jax import lax
from jax.experimental import pallas as pl
from jax.experimental.pallas import tpu as pltpu
from jax.experimental.pallas import tpu_sc as plsc

_NC = 2
_NS = 16
_NW = _NC * _NS
_L = 16
_UNROLL = 2


def _sc_affine_gather(x, idx, stab, ctab, *, chunk):
    n = x.shape[0]
    mesh = plsc.VectorSubcoreMesh(core_axis_name="c", subcore_axis_name="s")
    cp = pltpu.CompilerParams()
    if "needs_layout_passes" in pltpu.CompilerParams.__dataclass_fields__:
        cp = dataclasses.replace(cp, needs_layout_passes=False)

    @functools.partial(
        pl.kernel,
        out_type=jax.ShapeDtypeStruct((n,), jnp.float32),
        mesh=mesh,
        compiler_params=cp,
        scratch_types=[
            pltpu.VMEM((chunk,), jnp.int32),
            pltpu.VMEM((chunk,), jnp.float32),
            pltpu.VMEM((chunk,), jnp.float32),
            pltpu.VMEM(stab.shape, jnp.float32),
            pltpu.VMEM(ctab.shape, jnp.float32),
            pltpu.SemaphoreType.DMA,
            pltpu.SemaphoreType.DMA,
            pltpu.SemaphoreType.DMA,
            pltpu.SemaphoreType.DMA,
        ],
    )
    def body(x_hbm, idx_hbm, stab_hbm, ctab_hbm, out_hbm,
             idx_v, x_v, out_v, stab_v, ctab_v,
             sem0, sem1, sem2, sem3):
        wid = lax.axis_index("s") * _NC + lax.axis_index("c")
        base = jnp.where(wid == _NW - 1, n - chunk, wid * chunk)

        c0 = pltpu.async_copy(idx_hbm.at[pl.ds(base, chunk)], idx_v, sem0)
        c1 = pltpu.async_copy(x_hbm.at[pl.ds(base, chunk)], x_v, sem1)
        c2 = pltpu.async_copy(stab_hbm, stab_v, sem2)
        c3 = pltpu.async_copy(ctab_hbm, ctab_v, sem3)
        c0.wait()
        c1.wait()
        c2.wait()
        c3.wait()

        def vec(c):
            sl = pl.ds(c, _L)
            iv = idx_v[sl]
            xv = x_v[sl]
            s = plsc.load_gather(stab_v, [iv])
            sc = plsc.load_gather(ctab_v, [iv])
            out_v[sl] = s + sc * xv

        main = chunk - chunk % (_L * _UNROLL)

        @pl.loop(0, main, step=_L * _UNROLL)
        def _(c):
            for u in range(_UNROLL):
                vec(c + u * _L)

        @pl.loop(main, chunk, step=_L)
        def _(c):
            vec(c)

        pltpu.sync_copy(out_v, out_hbm.at[pl.ds(base, chunk)])

    return body(x, idx, stab, ctab)


def kernel(in_field, species_idx, shifts, scales):
    n = in_field.shape[0]
    x = in_field.reshape(n).astype(jnp.float32)
    idx = species_idx.astype(jnp.int32)
    stab = shifts.astype(jnp.float32)
    ctab = scales.astype(jnp.float32)

    chunk = ((n + _NW - 1) // _NW + _L - 1) // _L * _L
    assert chunk % _L == 0 and n % 8 == 0 and n >= chunk, (n, chunk)

    out = _sc_affine_gather(x, idx, stab, ctab, chunk=chunk)
    return out.reshape(n, 1)


# speedup vs baseline: 1.0306x; 1.0125x over previous
import dataclasses
import functools

import jax
import jax.numpy as jnp
from jax import lax
from jax.experimental import pallas as pl
from jax.experimental.pallas import tpu as pltpu
from jax.experimental.pallas import tpu_sc as plsc

_NC = 1
_NS = 16
_NW = _NC * _NS
_L = 16
_UNROLL = 2


def _sc_affine_gather(x, idx, stab, ctab, *, chunk):
    n = x.shape[0]
    mesh = plsc.VectorSubcoreMesh(core_axis_name="c", subcore_axis_name="s", num_cores=1)
    cp = pltpu.CompilerParams()
    if "needs_layout_passes" in pltpu.CompilerParams.__dataclass_fields__:
        cp = dataclasses.replace(cp, needs_layout_passes=False)

    @functools.partial(
        pl.kernel,
        out_type=jax.ShapeDtypeStruct((n,), jnp.float32),
        mesh=mesh,
        compiler_params=cp,
        scratch_types=[
            pltpu.VMEM((chunk,), jnp.int32),
            pltpu.VMEM((chunk,), jnp.float32),
            pltpu.VMEM((chunk,), jnp.float32),
            pltpu.VMEM(stab.shape, jnp.float32),
            pltpu.VMEM(ctab.shape, jnp.float32),
            pltpu.SemaphoreType.DMA,
            pltpu.SemaphoreType.DMA,
            pltpu.SemaphoreType.DMA,
            pltpu.SemaphoreType.DMA,
        ],
    )
    def body(x_hbm, idx_hbm, stab_hbm, ctab_hbm, out_hbm,
             idx_v, x_v, out_v, stab_v, ctab_v,
             sem0, sem1, sem2, sem3):
        wid = lax.axis_index("s") * _NC + lax.axis_index("c")
        base = jnp.where(wid == _NW - 1, n - chunk, wid * chunk)

        c0 = pltpu.async_copy(idx_hbm.at[pl.ds(base, chunk)], idx_v, sem0)
        c1 = pltpu.async_copy(x_hbm.at[pl.ds(base, chunk)], x_v, sem1)
        c2 = pltpu.async_copy(stab_hbm, stab_v, sem2)
        c3 = pltpu.async_copy(ctab_hbm, ctab_v, sem3)
        c0.wait()
        c1.wait()
        c2.wait()
        c3.wait()

        def vec(c):
            sl = pl.ds(c, _L)
            iv = idx_v[sl]
            xv = x_v[sl]
            s = plsc.load_gather(stab_v, [iv])
            sc = plsc.load_gather(ctab_v, [iv])
            out_v[sl] = s + sc * xv

        main = chunk - chunk % (_L * _UNROLL)

        @pl.loop(0, main, step=_L * _UNROLL)
        def _(c):
            for u in range(_UNROLL):
                vec(c + u * _L)

        @pl.loop(main, chunk, step=_L)
        def _(c):
            vec(c)

        pltpu.sync_copy(out_v, out_hbm.at[pl.ds(base, chunk)])

    return body(x, idx, stab, ctab)


def kernel(in_field, species_idx, shifts, scales):
    n = in_field.shape[0]
    x = in_field.reshape(n).astype(jnp.float32)
    idx = species_idx.astype(jnp.int32)
    stab = shifts.astype(jnp.float32)
    ctab = scales.astype(jnp.float32)

    chunk = ((n + _NW - 1) // _NW + _L - 1) // _L * _L
    assert chunk % _L == 0 and n % 8 == 0 and n >= chunk, (n, chunk)

    out = _sc_affine_gather(x, idx, stab, ctab, chunk=chunk)
    return out.reshape(n, 1)
